# baseline (device time: 12503 ns/iter reference)
import jax
import jax.numpy as jnp
from jax import lax
from jax.experimental import pallas as pl
from jax.experimental.pallas import tpu as pltpu

N_DEV = 8
B = 128


def kernel(x, w_mat):
    k, n_per = x.shape
    k_w, n = w_mat.shape

    def body(x_ref, w_ref, out_ref, xbf_ref, gather_ref,
             send_sems, recv_sems):
        my_pos = lax.axis_index("i")

        my_code = my_pos ^ ((my_pos >> 1) & 1)

        def partner(m):
            c = my_code ^ m
            return c ^ ((c >> 1) & 1)

        ORDER = (1, 2, 4, 3, 5, 6, 7)

        barrier_sem = pltpu.get_barrier_semaphore()
        for m in ORDER:
            pl.semaphore_signal(
                barrier_sem, inc=1,
                device_id=(partner(m),), device_id_type=pl.DeviceIdType.MESH,
            )

        xbf_ref[:, :] = x_ref[:, :].astype(jnp.bfloat16)

        pl.semaphore_wait(barrier_sem, N_DEV - 1)

        sends = []
        for m in ORDER:
            r = partner(m)
            rdma = pltpu.make_async_remote_copy(
                src_ref=xbf_ref.at[pl.ds(r * B, B), :],
                dst_ref=gather_ref.at[my_pos],
                send_sem=send_sems.at[m],
                recv_sem=recv_sems.at[my_pos],
                device_id=(r,),
                device_id_type=pl.DeviceIdType.MESH,
            )
            rdma.start()
            sends.append(rdma)

        y = jnp.dot(
            x_ref[pl.ds(my_pos * B, B), :],
            w_ref[pl.ds(my_pos * B, B), :],
            preferred_element_type=jnp.float32,
        )

        for m in ORDER:
            src = partner(m)
            recv = pltpu.make_async_remote_copy(
                src_ref=gather_ref.at[src],
                dst_ref=gather_ref.at[src],
                send_sem=send_sems.at[0],
                recv_sem=recv_sems.at[src],
                device_id=(src,),
                device_id_type=pl.DeviceIdType.MESH,
            )
            recv.wait_recv()
            y = y + jnp.dot(
                gather_ref[src],
                w_ref[pl.ds(src * B, B), :],
                preferred_element_type=jnp.float32,
            )

        for rdma in sends:
            rdma.wait_send()

        out_ref[:, :] = y * (1.0 / (1.0 + jnp.exp(-y)))

    return pl.pallas_call(
        body,
        out_shape=jax.ShapeDtypeStruct((B, n), jnp.float32),
        in_specs=[
            pl.BlockSpec(memory_space=pltpu.VMEM),
            pl.BlockSpec(memory_space=pltpu.VMEM),
        ],
        out_specs=pl.BlockSpec(memory_space=pltpu.VMEM),
        scratch_shapes=[
            pltpu.VMEM((k, n_per), jnp.bfloat16),
            pltpu.VMEM((N_DEV, B, n_per), jnp.bfloat16),
            pltpu.SemaphoreType.DMA((N_DEV,)),
            pltpu.SemaphoreType.DMA((N_DEV,)),
        ],
        compiler_params=pltpu.CompilerParams(collective_id=0),
    )(x, w_mat)


# device time: 11904 ns/iter; 1.0503x vs baseline; 1.0503x over previous
import jax
import jax.numpy as jnp
from jax import lax
from jax.experimental import pallas as pl
from jax.experimental.pallas import tpu as pltpu

N_DEV = 8
B = 128


def kernel(x, w_mat):
    k, n_per = x.shape
    k_w, n = w_mat.shape

    def body(x_ref, w_ref, out_ref, xbf_ref, row_ref, wperm_ref,
             send_sems, recv_sems):
        my_pos = lax.axis_index("i")

        my_code = my_pos ^ ((my_pos >> 1) & 1)

        def partner(m):
            c = my_code ^ m
            return c ^ ((c >> 1) & 1)

        ORDER = (1, 2, 4, 3, 5, 6, 7)

        barrier_sem = pltpu.get_barrier_semaphore()
        for m in ORDER:
            pl.semaphore_signal(
                barrier_sem, inc=1,
                device_id=(partner(m),), device_id_type=pl.DeviceIdType.MESH,
            )

        xbf_ref[:, :] = x_ref[:, :].astype(jnp.bfloat16)

        pl.semaphore_wait(barrier_sem, N_DEV - 1)

        sends = []
        for kk, m in enumerate(ORDER):
            r = partner(m)
            rdma = pltpu.make_async_remote_copy(
                src_ref=xbf_ref.at[pl.ds(r * B, B), :],
                dst_ref=row_ref.at[:, pl.ds(kk * B, B)],
                send_sem=send_sems.at[kk],
                recv_sem=recv_sems.at[kk],
                device_id=(r,),
                device_id_type=pl.DeviceIdType.MESH,
            )
            rdma.start()
            sends.append(rdma)

        row_ref[:, pl.ds(7 * B, B)] = xbf_ref[pl.ds(my_pos * B, B), :]
        for kk, m in enumerate(ORDER):
            src = partner(m)
            wperm_ref[kk * B:(kk + 1) * B, :] = (
                w_ref[pl.ds(src * B, B), :].astype(jnp.bfloat16))
        wperm_ref[7 * B:, :] = (
            w_ref[pl.ds(my_pos * B, B), :].astype(jnp.bfloat16))

        def wait_slot(kk):
            recv = pltpu.make_async_remote_copy(
                src_ref=xbf_ref.at[pl.ds(0, B), :],
                dst_ref=row_ref.at[:, pl.ds(kk * B, B)],
                send_sem=send_sems.at[kk],
                recv_sem=recv_sems.at[kk],
                device_id=(my_pos,),
                device_id_type=pl.DeviceIdType.MESH,
            )
            recv.wait_recv()

        y = jnp.zeros((B, n), dtype=jnp.float32)
        for slots in ((0, 1, 2), (3, 4, 5), (6,)):
            for kk in slots:
                wait_slot(kk)
            lo = slots[0] * B
            hi = (slots[-1] + 1) * B if slots != (6,) else N_DEV * B
            y = y + jnp.dot(
                row_ref[:, lo:hi],
                wperm_ref[lo:hi, :],
                preferred_element_type=jnp.float32,
            )

        for rdma in sends:
            rdma.wait_send()

        out_ref[:, :] = y * (1.0 / (1.0 + jnp.exp(-y)))

    return pl.pallas_call(
        body,
        out_shape=jax.ShapeDtypeStruct((B, n), jnp.float32),
        in_specs=[
            pl.BlockSpec(memory_space=pltpu.VMEM),
            pl.BlockSpec(memory_space=pltpu.VMEM),
        ],
        out_specs=pl.BlockSpec(memory_space=pltpu.VMEM),
        scratch_shapes=[
            pltpu.VMEM((k, n_per), jnp.bfloat16),
            pltpu.VMEM((B, k_w), jnp.bfloat16),
            pltpu.VMEM((k_w, n), jnp.bfloat16),
            pltpu.SemaphoreType.DMA((N_DEV,)),
            pltpu.SemaphoreType.DMA((N_DEV,)),
        ],
        compiler_params=pltpu.CompilerParams(collective_id=0),
    )(x, w_mat)
